# baseline (speedup 1.0000x reference)
import jax
import jax.numpy as jnp
from jax.experimental import pallas as pl
from jax.experimental.pallas import tpu as pltpu

B, T, H, A, E, L = 128, 512, 1024, 128, 16, 1000
GE = 4
NG = E // GE
KC = 256
NK = H // KC


def _adapter_head_kernel(lh_hbm, vids_ref, bd_ref, bu_ref, bc_ref,
                         Wd_hbm, Wu_hbm, Wc_hbm, Wd2_hbm, Wu2_hbm, Wc2_hbm,
                         out_ref,
                         x_buf, wd_buf, wu_buf, wc_buf,
                         x_sem, wd_sem, wu_sem, wc_sem):
    pltpu.make_async_copy(lh_hbm.at[:, 0], x_buf, x_sem).start()
    wd_refs = [Wd_hbm, Wd2_hbm]
    wu_refs = [Wu_hbm, Wu2_hbm]
    wc_refs = [Wc_hbm, Wc2_hbm]
    for g in range(NG):
        pltpu.make_async_copy(wd_refs[g % 2].at[pl.ds(g * GE, GE)],
                              wd_buf.at[g], wd_sem.at[g]).start()
        pltpu.make_async_copy(wu_refs[g % 2].at[pl.ds(g * GE, GE)],
                              wu_buf.at[g], wu_sem.at[g]).start()
    for k in range(NK):
        pltpu.make_async_copy(wc_refs[k % 2].at[pl.ds(k * KC, KC)],
                              wc_buf.at[k], wc_sem.at[k]).start()

    pltpu.make_async_copy(lh_hbm.at[:, 0], x_buf, x_sem).wait()
    x = x_buf[...]
    vids = vids_ref[...]
    iota = jax.lax.broadcasted_iota(jnp.int32, (B, E), 1)
    onehot = (vids == iota).astype(jnp.float32)
    bdg = jnp.dot(onehot, bd_ref[...],
                  preferred_element_type=jnp.float32)
    act = x + jnp.dot(onehot, bu_ref[...],
                      preferred_element_type=jnp.float32)
    for g in range(NG):
        pltpu.make_async_copy(wd_refs[g % 2].at[pl.ds(g * GE, GE)],
                              wd_buf.at[g], wd_sem.at[g]).wait()
        pltpu.make_async_copy(wu_refs[g % 2].at[pl.ds(g * GE, GE)],
                              wu_buf.at[g], wu_sem.at[g]).wait()
        for j in range(GE):
            e = g * GE + j
            m = (vids == e).astype(jnp.float32)
            h = jnp.dot(x, wd_buf[g, j], preferred_element_type=jnp.float32)
            h = jnp.maximum(h + bdg, 0.0) * m
            act = act + jnp.dot(h, wu_buf[g, j],
                                preferred_element_type=jnp.float32)

    acc = jnp.broadcast_to(bc_ref[...], (B, L))
    for k in range(NK):
        pltpu.make_async_copy(wc_refs[k % 2].at[pl.ds(k * KC, KC)],
                              wc_buf.at[k], wc_sem.at[k]).wait()
        acc = acc + jnp.dot(act[:, k * KC:(k + 1) * KC], wc_buf[k],
                            preferred_element_type=jnp.float32)
    out_ref[...] = acc


def kernel(last_hidden, attention_mask, variety_ids, W_down, b_down, W_up,
           b_up, W_c, b_c):
    vids = variety_ids.reshape(B, 1)
    logits = pl.pallas_call(
        _adapter_head_kernel,
        grid=(1,),
        in_specs=[
            pl.BlockSpec(memory_space=pltpu.MemorySpace.HBM),
            pl.BlockSpec((B, 1), lambda i: (0, 0)),
            pl.BlockSpec((E, A), lambda i: (0, 0)),
            pl.BlockSpec((E, H), lambda i: (0, 0)),
            pl.BlockSpec((1, L), lambda i: (0, 0)),
            pl.BlockSpec(memory_space=pltpu.MemorySpace.HBM),
            pl.BlockSpec(memory_space=pltpu.MemorySpace.HBM),
            pl.BlockSpec(memory_space=pltpu.MemorySpace.HBM),
            pl.BlockSpec(memory_space=pltpu.MemorySpace.HBM),
            pl.BlockSpec(memory_space=pltpu.MemorySpace.HBM),
            pl.BlockSpec(memory_space=pltpu.MemorySpace.HBM),
        ],
        out_specs=pl.BlockSpec((B, L), lambda i: (0, 0)),
        out_shape=jax.ShapeDtypeStruct((B, L), jnp.float32),
        scratch_shapes=[
            pltpu.VMEM((B, H), jnp.float32),
            pltpu.VMEM((NG, GE, H, A), jnp.float32),
            pltpu.VMEM((NG, GE, A, H), jnp.float32),
            pltpu.VMEM((NK, KC, L), jnp.float32),
            pltpu.SemaphoreType.DMA,
            pltpu.SemaphoreType.DMA((NG,)),
            pltpu.SemaphoreType.DMA((NG,)),
            pltpu.SemaphoreType.DMA((NK,)),
        ],
    )(last_hidden, vids, b_down, b_up, b_c.reshape(1, L), W_down, W_up, W_c,
      W_down, W_up, W_c)
    return logits


# speedup vs baseline: 1.0033x; 1.0033x over previous
import jax
import jax.numpy as jnp
from jax.experimental import pallas as pl
from jax.experimental.pallas import tpu as pltpu

B, T, H, A, E, L = 128, 512, 1024, 128, 16, 1000
GE = 4
NG = E // GE
KC = 256
NK = H // KC


def _adapter_head_kernel(lh_hbm, vids_ref, bd_ref, bu_ref, bc_ref,
                         Wd_hbm, Wu_hbm, Wc_hbm,
                         out_ref,
                         x_buf, wd_buf, wu_buf, wc_buf,
                         x_sem, wd_sem, wu_sem, wc_sem):
    pltpu.make_async_copy(lh_hbm.at[:, 0], x_buf, x_sem).start()
    for g in range(NG):
        pltpu.make_async_copy(Wd_hbm.at[pl.ds(g * GE, GE)],
                              wd_buf.at[g], wd_sem.at[g]).start()
        pltpu.make_async_copy(Wu_hbm.at[pl.ds(g * GE, GE)],
                              wu_buf.at[g], wu_sem.at[g]).start()
    for k in range(NK):
        pltpu.make_async_copy(Wc_hbm.at[pl.ds(k * KC, KC)],
                              wc_buf.at[k], wc_sem.at[k]).start()

    pltpu.make_async_copy(lh_hbm.at[:, 0], x_buf, x_sem).wait()
    x = x_buf[...]
    vids = vids_ref[...]
    iota = jax.lax.broadcasted_iota(jnp.int32, (B, E), 1)
    onehot = (vids == iota).astype(jnp.float32)
    bdg = jnp.dot(onehot, bd_ref[...],
                  preferred_element_type=jnp.float32)
    act = x + jnp.dot(onehot, bu_ref[...],
                      preferred_element_type=jnp.float32)
    for g in range(NG):
        pltpu.make_async_copy(Wd_hbm.at[pl.ds(g * GE, GE)],
                              wd_buf.at[g], wd_sem.at[g]).wait()
        pltpu.make_async_copy(Wu_hbm.at[pl.ds(g * GE, GE)],
                              wu_buf.at[g], wu_sem.at[g]).wait()
        for j in range(GE):
            e = g * GE + j
            m = (vids == e).astype(jnp.float32)
            h = jnp.dot(x, wd_buf[g, j], preferred_element_type=jnp.float32)
            h = jnp.maximum(h + bdg, 0.0) * m
            act = act + jnp.dot(h, wu_buf[g, j],
                                preferred_element_type=jnp.float32)

    acc = jnp.broadcast_to(bc_ref[...], (B, L))
    for k in range(NK):
        pltpu.make_async_copy(Wc_hbm.at[pl.ds(k * KC, KC)],
                              wc_buf.at[k], wc_sem.at[k]).wait()
        acc = acc + jnp.dot(act[:, k * KC:(k + 1) * KC], wc_buf[k],
                            preferred_element_type=jnp.float32)
    out_ref[...] = acc


def kernel(last_hidden, attention_mask, variety_ids, W_down, b_down, W_up,
           b_up, W_c, b_c):
    vids = variety_ids.reshape(B, 1)
    logits = pl.pallas_call(
        _adapter_head_kernel,
        grid=(1,),
        in_specs=[
            pl.BlockSpec(memory_space=pltpu.MemorySpace.HBM),
            pl.BlockSpec((B, 1), lambda i: (0, 0)),
            pl.BlockSpec((E, A), lambda i: (0, 0)),
            pl.BlockSpec((E, H), lambda i: (0, 0)),
            pl.BlockSpec((1, L), lambda i: (0, 0)),
            pl.BlockSpec(memory_space=pltpu.MemorySpace.HBM),
            pl.BlockSpec(memory_space=pltpu.MemorySpace.HBM),
            pl.BlockSpec(memory_space=pltpu.MemorySpace.HBM),
        ],
        out_specs=pl.BlockSpec((B, L), lambda i: (0, 0)),
        out_shape=jax.ShapeDtypeStruct((B, L), jnp.float32),
        scratch_shapes=[
            pltpu.VMEM((B, H), jnp.float32),
            pltpu.VMEM((NG, GE, H, A), jnp.float32),
            pltpu.VMEM((NG, GE, A, H), jnp.float32),
            pltpu.VMEM((NK, KC, L), jnp.float32),
            pltpu.SemaphoreType.DMA,
            pltpu.SemaphoreType.DMA((NG,)),
            pltpu.SemaphoreType.DMA((NG,)),
            pltpu.SemaphoreType.DMA((NK,)),
        ],
    )(last_hidden, vids, b_down, b_up, b_c.reshape(1, L), W_down, W_up, W_c)
    return logits
